# baseline (speedup 1.0000x reference)
import functools

import numpy as np
import jax
import jax.numpy as jnp
from jax.experimental import pallas as pl
from jax.experimental.pallas import tpu as pltpu
from jax.experimental.pallas import tpu_sc as plsc

DIM = 384
NH = 12
HD = DIM // NH
K = 7
NE = 8
HID = 768
B, H, W = 2, 32, 32
NSTRIP = 4
QR = 8
KR = 16
QT = QR * W
KT = KR * W
T = H * W

_STARTS = np.clip(np.arange(H) - K // 2, 0, H - K)
_KS = np.array([min(max(8 * s - 4, 0), H - KR) for s in range(NSTRIP)])


def _onehot_tables():
    s_ = np.arange(NSTRIP)[:, None, None]
    i_ = np.arange(QR)[None, :, None]
    j_ = np.arange(KR)[None, None, :]
    qr = 8 * s_ + i_
    kr = _KS[:, None, None] + j_
    qr, kr = np.broadcast_arrays(qr, kr)
    dr = kr - qr + (K - 1)
    rvalid = (kr >= _STARTS[qr]) & (kr < _STARTS[qr] + K)
    ridx = np.where(rvalid, np.clip(dr, 0, 2 * K - 2), 2 * K - 1)
    ohr = np.zeros((NSTRIP, QR, KR, 2 * K), np.float32)
    np.put_along_axis(ohr, ridx[..., None], 1.0, axis=-1)
    assert np.array_equal(ohr[1], ohr[2])
    ohr = ohr[[0, 1, 3]]
    qc = np.arange(W)[:, None]
    kc = np.arange(W)[None, :]
    dc = kc - qc + (K - 1)
    cvalid = (kc >= _STARTS[qc]) & (kc < _STARTS[qc] + K)
    cidx = np.where(cvalid, np.clip(dc, 0, 2 * K - 2), 2 * K - 1)
    ohc = np.zeros((W, W, 2 * K), np.float32)
    np.put_along_axis(ohc, cidx[..., None], 1.0, axis=-1)
    return ohr, ohc


_OHR, _OHC = _onehot_tables()


def _bias_tables(rpb):
    rpbe = jnp.full((NH, 2 * K, 2 * K), -1e9, jnp.float32)
    rpbe = rpbe.at[:, :2 * K - 1, :2 * K - 1].set(rpb)
    t2 = jnp.einsum('huv,qkv->huqk', rpbe, jnp.asarray(_OHC))
    tab = jnp.einsum('siju,huqk->shiqjk', jnp.asarray(_OHR), t2)
    return tab.reshape(3, NH, QT, KT).astype(jnp.bfloat16)


def _gelu(x):
    return 0.5 * x * (1.0 + jax.lax.erf(x * 0.7071067811865476))


def _ln(x, g, b):
    m = jnp.mean(x, axis=-1, keepdims=True)
    v = jnp.mean((x - m) ** 2, axis=-1, keepdims=True)
    return (x - m) * jax.lax.rsqrt(v + 1e-5) * g + b


def _attn_body(x_ref, ln1g_ref, ln1b_ref, wqkv_ref, bqkv_ref, bias_ref,
               wproj_ref, bproj_ref, ln2g_ref, ln2b_ref, wr_ref, br_ref,
               xout_ref, y_ref, ti_ref, tv_ref, qkv_scr):
    s = pl.program_id(0)
    ks = jnp.clip(8 * s - 4, 0, H - KR)
    xk = x_ref[0, pl.ds(ks, KR)].reshape(KT, DIM)
    xn = _ln(xk, ln1g_ref[...], ln1b_ref[...])
    qkv = jnp.dot(xn, wqkv_ref[...],
                  preferred_element_type=jnp.float32) + bqkv_ref[...]
    qo = 8 * s - ks
    qkv_scr[...] = qkv
    qrows = qkv_scr[pl.ds(qo * W, QT), :]
    scale = float(HD) ** -0.5
    outs = []
    for h in range(NH):
        qh = qrows[:, h * HD:(h + 1) * HD] * scale
        kh = qkv[:, DIM + h * HD:DIM + (h + 1) * HD]
        vh = qkv[:, 2 * DIM + h * HD:2 * DIM + (h + 1) * HD]
        sc = jax.lax.dot_general(qh, kh, (((1,), (1,)), ((), ())),
                                 preferred_element_type=jnp.float32)
        sc = sc + bias_ref[0, h].astype(jnp.float32)
        mx = jnp.max(sc, axis=-1, keepdims=True)
        p = jnp.exp(sc - mx)
        den = jnp.sum(p, axis=-1, keepdims=True)
        oh = jnp.dot(p, vh, preferred_element_type=jnp.float32) / den
        outs.append(oh)
    att = jnp.concatenate(outs, axis=1)
    proj = jnp.dot(att, wproj_ref[...],
                   preferred_element_type=jnp.float32) + bproj_ref[...]
    xa = x_ref[0, pl.ds(8 * s, QR)].reshape(QT, DIM) + proj
    xout_ref[0] = xa
    y = _ln(xa, ln2g_ref[...], ln2b_ref[...])
    y_ref[0] = y
    logits = jnp.dot(y, wr_ref[...],
                     preferred_element_type=jnp.float32) + br_ref[...]
    iota8 = jax.lax.broadcasted_iota(jnp.int32, (QT, NE), 1)
    m1 = jnp.max(logits, axis=-1, keepdims=True)
    i1 = jnp.min(jnp.where(logits >= m1, iota8, NE), axis=-1)
    l2m = jnp.where(iota8 == i1[:, None], -jnp.inf, logits)
    m2 = jnp.max(l2m, axis=-1, keepdims=True)
    i2 = jnp.min(jnp.where(l2m >= m2, iota8, NE), axis=-1)
    t = jnp.exp(m2[:, 0] - m1[:, 0])
    v1 = 1.0 / (1.0 + t)
    v2 = t / (1.0 + t)
    zi = jnp.zeros((NE - 2, QT), jnp.int32)
    zv = jnp.zeros((NE - 2, QT), jnp.float32)
    ti_ref[0] = jnp.concatenate([i1[None], i2[None], zi], axis=0)
    tv_ref[0] = jnp.concatenate([v1[None], v2[None], zv], axis=0)


NA = 2 * B * T
NT = NA // QT + NE
NSORT = NT * QT
_NW1 = 16
_APW = NA // _NW1
_NW2 = 32
_APW2 = NA // _NW2


def _route_body(topi_hbm, y_hbm, pos_hbm, ysort_hbm, tilee_hbm,
                keys_v, dest_v, tok_v, rows_v, stage_v, hist_v,
                base_ref, run_ref, gend_ref, tilee_v, hist_sh, sem):
    c = jax.lax.axis_index("c")
    s = jax.lax.axis_index("s")

    @pl.when(c == 0)
    def _work():
        w = s
        base_a = w * _APW
        lane = jax.lax.iota(jnp.int32, 16)
        pltpu.sync_copy(topi_hbm.at[pl.ds(base_a, _APW)], keys_v)
        counts = jnp.zeros((16,), jnp.int32)
        for r in range(_APW // 16):
            k = keys_v[pl.ds(16 * r, 16)]
            for e in range(NE):
                ce = plsc.all_reduce_population_count(k == e)
                counts = jnp.where(lane == e, counts + ce, counts)
        stage_v[...] = counts
        pltpu.sync_copy(stage_v, hist_sh.at[pl.ds(w * 16, 16)])
        plsc.subcore_barrier()
        pltpu.sync_copy(hist_sh, hist_v)
        tot = jnp.zeros((16,), jnp.int32)
        woff = jnp.zeros((16,), jnp.int32)
        for wp in range(_NW1):
            row = hist_v[pl.ds(wp * 16, 16)]
            tot = tot + row
            woff = woff + row * jnp.where(w > wp, 1, 0)
        ptot = ((tot + (QT - 1)) >> 8) << 8
        gend = plsc.cumsum(ptot)
        base = (gend - ptot) + woff
        base_ref[...] = base
        gend_ref[...] = gend
        runvec = jnp.zeros((16,), jnp.int32)
        for r in range(_APW // 16):
            k = keys_v[pl.ds(16 * r, 16)]
            run_ref[...] = runvec
            bg = plsc.load_gather(base_ref, [k])
            rg = plsc.load_gather(run_ref, [k])
            d = jnp.zeros((16,), jnp.int32)
            addc = jnp.zeros((16,), jnp.int32)
            for e in range(NE):
                m = k == e
                cs = plsc.cumsum(jnp.where(m, 1, 0))
                d = jnp.where(m, cs - 1, d)
                ce = plsc.all_reduce_population_count(m)
                addc = jnp.where(lane == e, addc + ce, addc)
            runvec = runvec + addc
            d = bg + rg + d
            dest_v[r // 8, pl.ds(16 * (r % 8), 16)] = d
            tok_v[r // 8, pl.ds(16 * (r % 8), 16)] = \
                (base_a + 16 * r + lane) >> 1
        pltpu.sync_copy(dest_v, pos_hbm.at[pl.ds(2 * w, 2)])
        for j in range(_APW // 128):
            pltpu.async_copy(y_hbm.at[tok_v.at[j]],
                             rows_v.at[pl.ds(128 * j, 128)], sem).wait()
            pltpu.async_copy(rows_v.at[pl.ds(128 * j, 128)],
                             ysort_hbm.at[dest_v.at[j]], sem).wait()

        @pl.when(w == 0)
        def _tiles():
            for half in range(NT // 16 + 1):
                t256 = (jax.lax.iota(jnp.int32, 16) + 16 * half) * QT
                acc = jnp.zeros((16,), jnp.int32)
                for e in range(NE):
                    ge = jnp.sum(jnp.where(lane == e, gend, 0))
                    acc = acc + jnp.where(t256 >= ge, 1, 0)
                tilee_v[pl.ds(16 * half, 16)] = jnp.minimum(acc, NE - 1)
            pltpu.sync_copy(tilee_v, tilee_hbm)


def _gatherback_body(eos_hbm, pos_hbm, out_hbm, idx_v, rows_v, sem):
    c = jax.lax.axis_index("c")
    s = jax.lax.axis_index("s")
    w = s * 2 + c
    pltpu.sync_copy(pos_hbm.at[w], idx_v)
    pltpu.async_copy(eos_hbm.at[idx_v], rows_v, sem).wait()
    pltpu.sync_copy(rows_v, out_hbm.at[pl.ds(w * _APW2, _APW2)])


def _expert_body(te_ref, ys_ref, we1_ref, be1_ref, we2_ref, be2_ref,
                 out_ref):
    h = jnp.dot(ys_ref[...], we1_ref[0],
                preferred_element_type=jnp.float32) + be1_ref[0, 0]
    out_ref[...] = jnp.dot(_gelu(h), we2_ref[0],
                           preferred_element_type=jnp.float32) + be2_ref[0, 0]


def _combine_body(xa_ref, y_ref, eop_ref, tv_ref, ws1_ref, bs1_ref,
                  ws2_ref, bs2_ref, out_ref):
    y = y_ref[0]
    v1 = tv_ref[0, 0, :]
    v2 = tv_ref[0, 1, :]
    hs = jnp.dot(y, ws1_ref[...],
                 preferred_element_type=jnp.float32) + bs1_ref[...]
    shared = jnp.dot(_gelu(hs), ws2_ref[...],
                     preferred_element_type=jnp.float32) + bs2_ref[...]
    eo = eop_ref[...]
    out_ref[0] = (xa_ref[0] + shared
                  + v1[:, None] * eo[:, :DIM] + v2[:, None] * eo[:, DIM:])


def kernel(x, ln1_g, ln1_b, ln2_g, ln2_b, W_qkv, b_qkv, rpb, W_proj, b_proj,
           W_r, b_r, W_e1, b_e1, W_e2, b_e2, W_s1, b_s1, W_s2, b_s2):
    bias_tab = _bias_tables(rpb)

    full = lambda *shape: pl.BlockSpec(shape, lambda s, b: (0,) * len(shape))
    attn_out = pl.pallas_call(
        _attn_body,
        grid=(NSTRIP, B),
        in_specs=[
            pl.BlockSpec((1, H, W, DIM), lambda s, b: (b, 0, 0, 0)),
            full(DIM), full(DIM),
            full(DIM, 3 * DIM), full(3 * DIM),
            pl.BlockSpec((1, NH, QT, KT),
                         lambda s, b: ((s > 0).astype(jnp.int32)
                                      + (s == 3).astype(jnp.int32), 0, 0, 0)),
            full(DIM, DIM), full(DIM),
            full(DIM), full(DIM),
            full(DIM, NE), full(NE),
        ],
        out_specs=[
            pl.BlockSpec((1, QT, DIM), lambda s, b: (b, s, 0)),
            pl.BlockSpec((1, QT, DIM), lambda s, b: (b, s, 0)),
            pl.BlockSpec((1, NE, QT), lambda s, b: (b, 0, s)),
            pl.BlockSpec((1, NE, QT), lambda s, b: (b, 0, s)),
        ],
        out_shape=[
            jax.ShapeDtypeStruct((B, T, DIM), jnp.float32),
            jax.ShapeDtypeStruct((B, T, DIM), jnp.float32),
            jax.ShapeDtypeStruct((B, NE, T), jnp.int32),
            jax.ShapeDtypeStruct((B, NE, T), jnp.float32),
        ],
        scratch_shapes=[pltpu.VMEM((KT, 3 * DIM), jnp.float32)],
    )(x, ln1_g, ln1_b, W_qkv, b_qkv, bias_tab, W_proj, b_proj,
      ln2_g, ln2_b, W_r, b_r)
    xa, y, ti, tv = attn_out

    topi_flat = jnp.stack([ti[:, 0, :], ti[:, 1, :]], axis=-1).reshape(NA)
    y2 = y.reshape(B * T, DIM)

    mesh = plsc.VectorSubcoreMesh(core_axis_name="c", subcore_axis_name="s",
                                  num_cores=2, num_subcores=16)
    pos2, y_sorted, tilee = pl.kernel(
        _route_body,
        out_type=[
            jax.ShapeDtypeStruct((2 * _NW1, 128), jnp.int32),
            jax.ShapeDtypeStruct((NSORT, DIM), jnp.float32),
            jax.ShapeDtypeStruct((NT + 8, ), jnp.int32),
        ],
        mesh=mesh,
        compiler_params=pltpu.CompilerParams(needs_layout_passes=False),
        scratch_types=[
            pltpu.VMEM((_APW,), jnp.int32),
            pltpu.VMEM((_APW // 128, 128), jnp.int32),
            pltpu.VMEM((_APW // 128, 128), jnp.int32),
            pltpu.VMEM((_APW, DIM), jnp.float32),
            pltpu.VMEM((16,), jnp.int32),
            pltpu.VMEM((_NW1 * 16,), jnp.int32),
            pltpu.VMEM((16,), jnp.int32),
            pltpu.VMEM((16,), jnp.int32),
            pltpu.VMEM((16,), jnp.int32),
            pltpu.VMEM((NT + 8,), jnp.int32),
            pltpu.VMEM_SHARED((_NW1 * 16,), jnp.int32),
            pltpu.SemaphoreType.DMA,
        ],
    )(topi_flat, y2)

    eo_sorted = pl.pallas_call(
        _expert_body,
        grid_spec=pltpu.PrefetchScalarGridSpec(
            num_scalar_prefetch=1,
            grid=(NT,),
            in_specs=[
                pl.BlockSpec((QT, DIM), lambda i, te: (i, 0)),
                pl.BlockSpec((1, DIM, HID), lambda i, te: (te[i], 0, 0)),
                pl.BlockSpec((1, 1, HID), lambda i, te: (te[i], 0, 0)),
                pl.BlockSpec((1, HID, DIM), lambda i, te: (te[i], 0, 0)),
                pl.BlockSpec((1, 1, DIM), lambda i, te: (te[i], 0, 0)),
            ],
            out_specs=pl.BlockSpec((QT, DIM), lambda i, te: (i, 0)),
        ),
        out_shape=jax.ShapeDtypeStruct((NSORT, DIM), jnp.float32),
    )(tilee, y_sorted, W_e1, b_e1.reshape(NE, 1, HID),
      W_e2, b_e2.reshape(NE, 1, DIM))

    eo_pairs = pl.kernel(
        _gatherback_body,
        out_type=jax.ShapeDtypeStruct((NA, DIM), jnp.float32),
        mesh=mesh,
        compiler_params=pltpu.CompilerParams(needs_layout_passes=False),
        scratch_types=[
            pltpu.VMEM((_APW2,), jnp.int32),
            pltpu.VMEM((_APW2, DIM), jnp.float32),
            pltpu.SemaphoreType.DMA,
        ],
    )(eo_sorted, pos2)

    fullm = lambda *shape: pl.BlockSpec(shape, lambda i: (0,) * len(shape))
    out = pl.pallas_call(
        _combine_body,
        grid=(B * NSTRIP,),
        in_specs=[
            pl.BlockSpec((1, QT, DIM), lambda i: (i // NSTRIP, i % NSTRIP, 0)),
            pl.BlockSpec((1, QT, DIM), lambda i: (i // NSTRIP, i % NSTRIP, 0)),
            pl.BlockSpec((QT, 2 * DIM), lambda i: (i, 0)),
            pl.BlockSpec((1, NE, QT), lambda i: (i // NSTRIP, 0, i % NSTRIP)),
            fullm(DIM, HID), fullm(HID),
            fullm(HID, DIM), fullm(DIM),
        ],
        out_specs=pl.BlockSpec((1, QT, DIM),
                               lambda i: (i // NSTRIP, i % NSTRIP, 0)),
        out_shape=jax.ShapeDtypeStruct((B, T, DIM), jnp.float32),
    )(xa, y, eo_pairs.reshape(B * T, 2 * DIM), tv, W_s1, b_s1, W_s2, b_s2)
    return out.reshape(B, H, W, DIM)
